# baseline (speedup 1.0000x reference)
import functools

import jax
import jax.numpy as jnp
from jax.experimental import pallas as pl
from jax.experimental.pallas import tpu as pltpu

_NEG = -1e30


def _cgmn_body(x_ref, b_ref, prior_ref, em_ref, cmT_ref, ghWT_ref, ghb_ref,
               goW_ref, gob_ref, outWT_ref, outb_ref, out_ref,
               T_s, m_s, d_s, r_s, *, n_tiles, n_graphs):
    i = pl.program_id(0)
    G, M = T_s.shape
    P = r_s.shape[0]
    TN = x_ref.shape[2]

    @pl.when(i == 0)
    def _init():
        pr = prior_ref[...]
        pe = jnp.exp(pr - jnp.max(pr, axis=1, keepdims=True))
        p = pe / jnp.sum(pe, axis=1, keepdims=True)
        em = em_ref[...]
        ee = jnp.exp(em - jnp.max(em, axis=2, keepdims=True))
        ep = ee / jnp.sum(ee, axis=2, keepdims=True)
        T_s[...] = jnp.log(jnp.sum(p[:, :, None] * ep, axis=1))
        m_s[...] = jnp.full((n_graphs, 1), _NEG, jnp.float32)
        d_s[...] = jnp.zeros((n_graphs, 1), jnp.float32)
        r_s[...] = jnp.zeros((P, n_graphs), jnp.float32)

    xi = x_ref[0]
    bi = b_ref[0]

    miota = jax.lax.broadcasted_iota(jnp.int32, (M, TN), 0)
    ohM = (miota == xi).astype(jnp.float32)
    llT = jnp.dot(T_s[...], ohM, preferred_element_type=jnp.float32)

    ciT = jnp.tanh(jnp.dot(cmT_ref[...], llT,
                           preferred_element_type=jnp.float32))
    h = jnp.tanh(jnp.dot(ghWT_ref[...], ciT,
                         preferred_element_type=jnp.float32) + ghb_ref[...])
    gate = jnp.sum(h * goW_ref[...], axis=0, keepdims=True) + gob_ref[...]

    giota = jax.lax.broadcasted_iota(jnp.int32, (n_graphs, TN), 0)
    ohG = giota == bi
    ohGf = ohG.astype(jnp.float32)
    tmax = jnp.max(jnp.where(ohG, gate, _NEG), axis=1, keepdims=True)
    m_old = m_s[...]
    m_new = jnp.maximum(m_old, tmax)
    scale = jnp.exp(m_old - m_new)
    mb = jnp.sum(ohGf * m_new, axis=0, keepdims=True)
    e = jnp.exp(gate - mb)
    ohGe = ohGf * e
    d_s[...] = d_s[...] * scale + jnp.sum(ohGe, axis=1, keepdims=True)
    rtile = jax.lax.dot_general(ciT, ohGe, (((1,), (1,)), ((), ())),
                                preferred_element_type=jnp.float32)
    eyeG = (jax.lax.broadcasted_iota(jnp.int32, (n_graphs, n_graphs), 0) ==
            jax.lax.broadcasted_iota(jnp.int32, (n_graphs, n_graphs), 1))
    sdiag = jnp.where(eyeG, scale, 0.0)
    r_s[...] = jnp.dot(r_s[...], sdiag,
                       preferred_element_type=jnp.float32) + rtile
    m_s[...] = m_new

    @pl.when(i == n_tiles - 1)
    def _fin():
        ddiag = jnp.where(eyeG, 1.0 / (d_s[...] + 1e-16), 0.0)
        rdiv = jnp.dot(r_s[...], ddiag,
                       preferred_element_type=jnp.float32)
        out_ref[...] = jnp.dot(outWT_ref[...], rdiv,
                               preferred_element_type=jnp.float32) + outb_ref[...]


def kernel(x, edge_index, batch, prior, emission, gh_W, gh_b, go_W, go_b,
           out_W, out_b, contrastive):
    del edge_index
    N = x.shape[0]
    G, C = prior.shape
    M = emission.shape[2]
    P = contrastive.shape[1]
    H = gh_W.shape[1]
    F = out_W.shape[1]
    NG = 64

    TN = 1024
    NT = -(-N // TN)
    Npad = NT * TN

    x32 = x.astype(jnp.int32)
    b32 = batch.astype(jnp.int32)
    xp = jnp.concatenate([x32, jnp.zeros((Npad - N,), jnp.int32)])
    bp = jnp.concatenate([b32, jnp.full((Npad - N,), NG, jnp.int32)])
    x3 = xp.reshape(NT, 1, TN)
    b3 = bp.reshape(NT, 1, TN)

    f32 = jnp.float32
    body = functools.partial(_cgmn_body, n_tiles=NT, n_graphs=NG)
    out = pl.pallas_call(
        body,
        grid=(NT,),
        in_specs=[
            pl.BlockSpec((1, 1, TN), lambda i: (i, 0, 0)),
            pl.BlockSpec((1, 1, TN), lambda i: (i, 0, 0)),
            pl.BlockSpec((G, C), lambda i: (0, 0)),
            pl.BlockSpec((G, C, M), lambda i: (0, 0, 0)),
            pl.BlockSpec((P, G), lambda i: (0, 0)),
            pl.BlockSpec((H, P), lambda i: (0, 0)),
            pl.BlockSpec((H, 1), lambda i: (0, 0)),
            pl.BlockSpec((H, 1), lambda i: (0, 0)),
            pl.BlockSpec((1, 1), lambda i: (0, 0)),
            pl.BlockSpec((F, P), lambda i: (0, 0)),
            pl.BlockSpec((F, 1), lambda i: (0, 0)),
        ],
        out_specs=pl.BlockSpec((F, NG), lambda i: (0, 0)),
        out_shape=jax.ShapeDtypeStruct((F, NG), f32),
        scratch_shapes=[
            pltpu.VMEM((G, M), f32),
            pltpu.VMEM((NG, 1), f32),
            pltpu.VMEM((NG, 1), f32),
            pltpu.VMEM((P, NG), f32),
        ],
    )(x3, b3,
      prior.astype(f32),
      emission.astype(f32),
      contrastive.T.astype(f32),
      gh_W.T.astype(f32),
      gh_b.reshape(H, 1).astype(f32),
      go_W.reshape(H, 1).astype(f32),
      go_b.reshape(1, 1).astype(f32),
      out_W.T.astype(f32),
      out_b.reshape(F, 1).astype(f32))
    return out.T


# speedup vs baseline: 1.1078x; 1.1078x over previous
import functools

import jax
import jax.numpy as jnp
from jax.experimental import pallas as pl
from jax.experimental.pallas import tpu as pltpu

_NEG = -1e30


def _cgmn_body(x_ref, b_ref, bc_ref, prior_ref, em_ref, cmT_ref, ghWT_ref,
               ghb_ref, goW_ref, gob_ref, outWT_ref, outb_ref, out_ref,
               T_s, m_s, d_s, r_s, *, n_tiles, n_graphs):
    i = pl.program_id(0)
    G, M = T_s.shape
    P = r_s.shape[0]
    TN = x_ref.shape[2]

    @pl.when(i == 0)
    def _init():
        pr = prior_ref[...]
        pe = jnp.exp(pr - jnp.max(pr, axis=1, keepdims=True))
        p = pe / jnp.sum(pe, axis=1, keepdims=True)
        em = em_ref[...]
        ee = jnp.exp(em - jnp.max(em, axis=2, keepdims=True))
        ep = ee / jnp.sum(ee, axis=2, keepdims=True)
        T_s[...] = jnp.log(jnp.sum(p[:, :, None] * ep, axis=1))
        m_s[...] = jnp.full((n_graphs, 1), _NEG, jnp.float32)
        d_s[...] = jnp.zeros((n_graphs, 1), jnp.float32)
        r_s[...] = jnp.zeros((P, n_graphs), jnp.float32)

    xi = x_ref[0]
    bi = b_ref[0]

    miota = jax.lax.broadcasted_iota(jnp.int32, (M, TN), 0)
    ohM = (miota == xi).astype(jnp.float32)
    llT = jnp.dot(T_s[...], ohM, preferred_element_type=jnp.float32)

    ciT = jnp.tanh(jnp.dot(cmT_ref[...], llT,
                           preferred_element_type=jnp.float32))
    h = jnp.tanh(jnp.dot(ghWT_ref[...], ciT,
                         preferred_element_type=jnp.float32) + ghb_ref[...])
    gate = jnp.sum(h * goW_ref[...], axis=0, keepdims=True) + gob_ref[...]

    giota = jax.lax.broadcasted_iota(jnp.int32, (n_graphs, TN), 0)
    ohG = giota == bi
    ohGf = ohG.astype(jnp.float32)
    tmax = jnp.max(jnp.where(ohG, gate, _NEG), axis=1, keepdims=True)
    m_old = m_s[...]
    m_new = jnp.maximum(m_old, tmax)
    scale = jnp.exp(m_old - m_new)
    mb = jnp.sum(ohGf * m_new, axis=0, keepdims=True)
    e = jnp.exp(gate - mb)
    ohGe = ohGf * e
    d_s[...] = d_s[...] * scale + jnp.sum(ohGe, axis=1, keepdims=True)
    ecol = jnp.transpose(e)
    bcol = bc_ref[0]
    ohGT = (bcol == jax.lax.broadcasted_iota(jnp.int32, (TN, n_graphs), 1))
    ohGeT = ohGT.astype(jnp.float32) * ecol
    rtile = jnp.dot(ciT, ohGeT, preferred_element_type=jnp.float32)
    scale_row = jnp.transpose(scale)
    r_s[...] = r_s[...] * scale_row + rtile
    m_s[...] = m_new

    @pl.when(i == n_tiles - 1)
    def _fin():
        drow = jnp.transpose(d_s[...])
        rdiv = r_s[...] / (drow + 1e-16)
        out_ref[...] = jnp.dot(outWT_ref[...], rdiv,
                               preferred_element_type=jnp.float32) + outb_ref[...]


def kernel(x, edge_index, batch, prior, emission, gh_W, gh_b, go_W, go_b,
           out_W, out_b, contrastive):
    del edge_index
    N = x.shape[0]
    G, C = prior.shape
    M = emission.shape[2]
    P = contrastive.shape[1]
    H = gh_W.shape[1]
    F = out_W.shape[1]
    NG = 64

    TN = 1024
    NT = -(-N // TN)
    Npad = NT * TN

    x32 = x.astype(jnp.int32)
    b32 = batch.astype(jnp.int32)
    xp = jnp.concatenate([x32, jnp.zeros((Npad - N,), jnp.int32)])
    bp = jnp.concatenate([b32, jnp.full((Npad - N,), NG, jnp.int32)])
    x3 = xp.reshape(NT, 1, TN)
    b3 = bp.reshape(NT, 1, TN)
    bc3 = bp.reshape(NT, TN, 1)

    f32 = jnp.float32
    body = functools.partial(_cgmn_body, n_tiles=NT, n_graphs=NG)
    out = pl.pallas_call(
        body,
        grid=(NT,),
        in_specs=[
            pl.BlockSpec((1, 1, TN), lambda i: (i, 0, 0)),
            pl.BlockSpec((1, 1, TN), lambda i: (i, 0, 0)),
            pl.BlockSpec((1, TN, 1), lambda i: (i, 0, 0)),
            pl.BlockSpec((G, C), lambda i: (0, 0)),
            pl.BlockSpec((G, C, M), lambda i: (0, 0, 0)),
            pl.BlockSpec((P, G), lambda i: (0, 0)),
            pl.BlockSpec((H, P), lambda i: (0, 0)),
            pl.BlockSpec((H, 1), lambda i: (0, 0)),
            pl.BlockSpec((H, 1), lambda i: (0, 0)),
            pl.BlockSpec((1, 1), lambda i: (0, 0)),
            pl.BlockSpec((F, P), lambda i: (0, 0)),
            pl.BlockSpec((F, 1), lambda i: (0, 0)),
        ],
        out_specs=pl.BlockSpec((F, NG), lambda i: (0, 0)),
        out_shape=jax.ShapeDtypeStruct((F, NG), f32),
        scratch_shapes=[
            pltpu.VMEM((G, M), f32),
            pltpu.VMEM((NG, 1), f32),
            pltpu.VMEM((NG, 1), f32),
            pltpu.VMEM((P, NG), f32),
        ],
    )(x3, b3, bc3,
      prior.astype(f32),
      emission.astype(f32),
      contrastive.T.astype(f32),
      gh_W.T.astype(f32),
      gh_b.reshape(H, 1).astype(f32),
      go_W.reshape(H, 1).astype(f32),
      go_b.reshape(1, 1).astype(f32),
      out_W.T.astype(f32),
      out_b.reshape(F, 1).astype(f32))
    return out.T


# speedup vs baseline: 1.6572x; 1.4959x over previous
import functools

import jax
import jax.numpy as jnp
from jax.experimental import pallas as pl
from jax.experimental.pallas import tpu as pltpu

_NEG = -1e30


def _cgmn_body(x_ref, b_ref, prior_ref, em_ref, cmT_ref, ghWT_ref,
               ghb_ref, goW_ref, gob_ref, outWT_ref, outb_ref, out_ref,
               T_s, m_s, d_s, num_s, *, n_tiles, n_graphs):
    i = pl.program_id(0)
    G, M = T_s.shape
    F = num_s.shape[1]
    TN = x_ref.shape[2]

    @pl.when(i == 0)
    def _init():
        pr = prior_ref[...]
        pe = jnp.exp(pr - jnp.max(pr, axis=1, keepdims=True))
        p = pe / jnp.sum(pe, axis=1, keepdims=True)
        em = em_ref[...]
        ee = jnp.exp(em - jnp.max(em, axis=2, keepdims=True))
        ep = ee / jnp.sum(ee, axis=2, keepdims=True)
        T_s[...] = jnp.log(jnp.sum(p[:, :, None] * ep, axis=1))
        m_s[...] = jnp.full((n_graphs, 1), _NEG, jnp.float32)
        d_s[...] = jnp.zeros((n_graphs, 1), jnp.float32)
        num_s[...] = jnp.zeros((n_graphs, F), jnp.float32)

    xi = x_ref[0]
    bi = b_ref[0]

    miota = jax.lax.broadcasted_iota(jnp.int32, (M, TN), 0)
    ohM = (miota == xi).astype(jnp.float32)
    llT = jnp.dot(T_s[...], ohM, preferred_element_type=jnp.float32)

    ciT = jnp.tanh(jnp.dot(cmT_ref[...], llT,
                           preferred_element_type=jnp.float32))
    h = jnp.tanh(jnp.dot(ghWT_ref[...], ciT,
                         preferred_element_type=jnp.float32) + ghb_ref[...])
    gate = jnp.sum(h * goW_ref[...], axis=0, keepdims=True) + gob_ref[...]

    giota = jax.lax.broadcasted_iota(jnp.int32, (n_graphs, TN), 0)
    ohG = giota == bi
    ohGf = ohG.astype(jnp.float32)
    tmax = jnp.max(jnp.where(ohG, gate, _NEG), axis=1, keepdims=True)
    m_old = m_s[...]
    m_new = jnp.maximum(m_old, tmax)
    scale = jnp.exp(m_old - m_new)
    mb = jnp.sum(ohGf * m_new, axis=0, keepdims=True)
    e = jnp.exp(gate - mb)
    ohGe = ohGf * e
    d_s[...] = d_s[...] * scale + jnp.sum(ohGe, axis=1, keepdims=True)
    vT = jnp.dot(outWT_ref[...], ciT,
                 preferred_element_type=jnp.float32)
    numtile = jax.lax.dot_general(ohGe, vT, (((1,), (1,)), ((), ())),
                                  preferred_element_type=jnp.float32)
    num_s[...] = num_s[...] * scale + numtile
    m_s[...] = m_new

    @pl.when(i == n_tiles - 1)
    def _fin():
        out_ref[...] = num_s[...] / (d_s[...] + 1e-16) + outb_ref[...]


def kernel(x, edge_index, batch, prior, emission, gh_W, gh_b, go_W, go_b,
           out_W, out_b, contrastive):
    del edge_index
    N = x.shape[0]
    G, C = prior.shape
    M = emission.shape[2]
    P = contrastive.shape[1]
    H = gh_W.shape[1]
    F = out_W.shape[1]
    NG = 64

    TN = 1024
    NT = -(-N // TN)
    Npad = NT * TN

    x32 = x.astype(jnp.int32)
    b32 = batch.astype(jnp.int32)
    xp = jnp.concatenate([x32, jnp.zeros((Npad - N,), jnp.int32)])
    bp = jnp.concatenate([b32, jnp.full((Npad - N,), NG, jnp.int32)])
    x3 = xp.reshape(NT, 1, TN)
    b3 = bp.reshape(NT, 1, TN)

    f32 = jnp.float32
    body = functools.partial(_cgmn_body, n_tiles=NT, n_graphs=NG)
    out = pl.pallas_call(
        body,
        grid=(NT,),
        in_specs=[
            pl.BlockSpec((1, 1, TN), lambda i: (i, 0, 0)),
            pl.BlockSpec((1, 1, TN), lambda i: (i, 0, 0)),
            pl.BlockSpec((G, C), lambda i: (0, 0)),
            pl.BlockSpec((G, C, M), lambda i: (0, 0, 0)),
            pl.BlockSpec((P, G), lambda i: (0, 0)),
            pl.BlockSpec((H, P), lambda i: (0, 0)),
            pl.BlockSpec((H, 1), lambda i: (0, 0)),
            pl.BlockSpec((H, 1), lambda i: (0, 0)),
            pl.BlockSpec((1, 1), lambda i: (0, 0)),
            pl.BlockSpec((F, P), lambda i: (0, 0)),
            pl.BlockSpec((1, F), lambda i: (0, 0)),
        ],
        out_specs=pl.BlockSpec((NG, F), lambda i: (0, 0)),
        out_shape=jax.ShapeDtypeStruct((NG, F), f32),
        scratch_shapes=[
            pltpu.VMEM((G, M), f32),
            pltpu.VMEM((NG, 1), f32),
            pltpu.VMEM((NG, 1), f32),
            pltpu.VMEM((NG, F), f32),
        ],
    )(x3, b3,
      prior.astype(f32),
      emission.astype(f32),
      contrastive.T.astype(f32),
      gh_W.T.astype(f32),
      gh_b.reshape(H, 1).astype(f32),
      go_W.reshape(H, 1).astype(f32),
      go_b.reshape(1, 1).astype(f32),
      out_W.T.astype(f32),
      out_b.reshape(1, F).astype(f32))
    return out


# speedup vs baseline: 1.6704x; 1.0080x over previous
import functools

import jax
import jax.numpy as jnp
from jax.experimental import pallas as pl
from jax.experimental.pallas import tpu as pltpu

_NEG = -1e30


def _cgmn_body(x_ref, b_ref, prior_ref, em_ref, cmT_ref, ghWT_ref,
               ghb_ref, goW_ref, gob_ref, outWT_ref, outb_ref, out_ref,
               T_s, m_s, d_s, num_s, *, n_tiles, n_graphs):
    i = pl.program_id(0)
    G, M = T_s.shape
    F = num_s.shape[1]
    TN = x_ref.shape[2]

    @pl.when(i == 0)
    def _init():
        pr = prior_ref[...]
        pe = jnp.exp(pr - jnp.max(pr, axis=1, keepdims=True))
        p = pe / jnp.sum(pe, axis=1, keepdims=True)
        em = em_ref[...]
        ee = jnp.exp(em - jnp.max(em, axis=2, keepdims=True))
        ep = ee / jnp.sum(ee, axis=2, keepdims=True)
        T_s[...] = jnp.log(jnp.sum(p[:, :, None] * ep, axis=1))
        m_s[...] = jnp.full((n_graphs, 1), _NEG, jnp.float32)
        d_s[...] = jnp.zeros((n_graphs, 1), jnp.float32)
        num_s[...] = jnp.zeros((n_graphs, F), jnp.float32)

    xi = x_ref[0]
    bi = b_ref[0]

    miota = jax.lax.broadcasted_iota(jnp.int32, (M, TN), 0)
    ohM = (miota == xi).astype(jnp.float32)
    llT = jnp.dot(T_s[...], ohM, preferred_element_type=jnp.float32)

    ciT = jnp.tanh(jnp.dot(cmT_ref[...], llT,
                           preferred_element_type=jnp.float32))
    h = jnp.tanh(jnp.dot(ghWT_ref[...], ciT,
                         preferred_element_type=jnp.float32) + ghb_ref[...])
    gate = jnp.sum(h * goW_ref[...], axis=0, keepdims=True) + gob_ref[...]

    giota = jax.lax.broadcasted_iota(jnp.int32, (n_graphs, TN), 0)
    ohG = giota == bi
    ohGf = ohG.astype(jnp.float32)
    tmax = jnp.max(jnp.where(ohG, gate, _NEG), axis=1, keepdims=True)
    m_old = m_s[...]
    m_new = jnp.maximum(m_old, tmax)
    scale = jnp.exp(m_old - m_new)
    mb = jnp.sum(ohGf * m_new, axis=0, keepdims=True)
    e = jnp.exp(gate - mb)
    ohGe = ohGf * e
    d_s[...] = d_s[...] * scale + jnp.sum(ohGe, axis=1, keepdims=True)
    vT = jnp.dot(outWT_ref[...], ciT,
                 preferred_element_type=jnp.float32)
    numtile = jax.lax.dot_general(ohGe, vT, (((1,), (1,)), ((), ())),
                                  preferred_element_type=jnp.float32)
    num_s[...] = num_s[...] * scale + numtile
    m_s[...] = m_new

    @pl.when(i == n_tiles - 1)
    def _fin():
        out_ref[...] = num_s[...] / (d_s[...] + 1e-16) + outb_ref[...]


def kernel(x, edge_index, batch, prior, emission, gh_W, gh_b, go_W, go_b,
           out_W, out_b, contrastive):
    del edge_index
    N = x.shape[0]
    G, C = prior.shape
    M = emission.shape[2]
    P = contrastive.shape[1]
    H = gh_W.shape[1]
    F = out_W.shape[1]
    NG = 64

    TN = 1000
    assert N % TN == 0
    NT = N // TN

    x3 = x.astype(jnp.int32).reshape(NT, 1, TN)
    b3 = batch.astype(jnp.int32).reshape(NT, 1, TN)

    f32 = jnp.float32
    body = functools.partial(_cgmn_body, n_tiles=NT, n_graphs=NG)
    out = pl.pallas_call(
        body,
        grid=(NT,),
        in_specs=[
            pl.BlockSpec((1, 1, TN), lambda i: (i, 0, 0)),
            pl.BlockSpec((1, 1, TN), lambda i: (i, 0, 0)),
            pl.BlockSpec((G, C), lambda i: (0, 0)),
            pl.BlockSpec((G, C, M), lambda i: (0, 0, 0)),
            pl.BlockSpec((P, G), lambda i: (0, 0)),
            pl.BlockSpec((H, P), lambda i: (0, 0)),
            pl.BlockSpec((H, 1), lambda i: (0, 0)),
            pl.BlockSpec((H, 1), lambda i: (0, 0)),
            pl.BlockSpec((1, 1), lambda i: (0, 0)),
            pl.BlockSpec((F, P), lambda i: (0, 0)),
            pl.BlockSpec((1, F), lambda i: (0, 0)),
        ],
        out_specs=pl.BlockSpec((NG, F), lambda i: (0, 0)),
        out_shape=jax.ShapeDtypeStruct((NG, F), f32),
        scratch_shapes=[
            pltpu.VMEM((G, M), f32),
            pltpu.VMEM((NG, 1), f32),
            pltpu.VMEM((NG, 1), f32),
            pltpu.VMEM((NG, F), f32),
        ],
    )(x3, b3,
      prior.astype(f32),
      emission.astype(f32),
      contrastive.T.astype(f32),
      gh_W.T.astype(f32),
      gh_b.reshape(H, 1).astype(f32),
      go_W.reshape(H, 1).astype(f32),
      go_b.reshape(1, 1).astype(f32),
      out_W.T.astype(f32),
      out_b.reshape(1, F).astype(f32))
    return out


# speedup vs baseline: 1.7567x; 1.0517x over previous
import functools

import jax
import jax.numpy as jnp
from jax.experimental import pallas as pl
from jax.experimental.pallas import tpu as pltpu

_NEG = -1e30


def _cgmn_body(x_ref, b_ref, prior_ref, em_ref, cmT_ref, ghWT_ref,
               ghb_ref, goW_ref, gob_ref, outWT_ref, outb_ref, out_ref,
               T_s, m_s, d_s, num_s, *, n_tiles, n_graphs):
    i = pl.program_id(0)
    G, M = T_s.shape
    F = num_s.shape[1]
    TN = x_ref.shape[2]

    @pl.when(i == 0)
    def _init():
        pr = prior_ref[...]
        pe = jnp.exp(pr - jnp.max(pr, axis=1, keepdims=True))
        p = pe / jnp.sum(pe, axis=1, keepdims=True)
        em = em_ref[...]
        ee = jnp.exp(em - jnp.max(em, axis=2, keepdims=True))
        ep = ee / jnp.sum(ee, axis=2, keepdims=True)
        T_s[...] = jnp.log(jnp.sum(p[:, :, None] * ep, axis=1))
        m_s[...] = jnp.full((n_graphs, 1), _NEG, jnp.float32)
        d_s[...] = jnp.zeros((n_graphs, 1), jnp.float32)
        num_s[...] = jnp.zeros((n_graphs, F), jnp.float32)

    xi = x_ref[0]
    bi = b_ref[0]

    miota = jax.lax.broadcasted_iota(jnp.int32, (M, TN), 0)
    ohM = (miota == xi).astype(jnp.float32)
    llT = jnp.dot(T_s[...], ohM, preferred_element_type=jnp.float32)

    ciT = jnp.tanh(jnp.dot(cmT_ref[...], llT,
                           preferred_element_type=jnp.float32))
    h = jnp.tanh(jnp.dot(ghWT_ref[...], ciT,
                         preferred_element_type=jnp.float32) + ghb_ref[...])
    gate = jnp.sum(h * goW_ref[...], axis=0, keepdims=True) + gob_ref[...]

    giota = jax.lax.broadcasted_iota(jnp.int32, (n_graphs, TN), 0)
    ohG = giota == bi
    ohGf = ohG.astype(jnp.float32)
    tmax = jnp.max(jnp.where(ohG, gate, _NEG), axis=1, keepdims=True)
    m_old = m_s[...]
    m_new = jnp.maximum(m_old, tmax)
    scale = jnp.exp(m_old - m_new)
    mb = jnp.sum(ohGf * m_new, axis=0, keepdims=True)
    e = jnp.exp(gate - mb)
    ohGe = ohGf * e
    d_s[...] = d_s[...] * scale + jnp.sum(ohGe, axis=1, keepdims=True)
    vT = jnp.dot(outWT_ref[...], ciT,
                 preferred_element_type=jnp.float32)
    numtile = jax.lax.dot_general(ohGe, vT, (((1,), (1,)), ((), ())),
                                  preferred_element_type=jnp.float32)
    num_s[...] = num_s[...] * scale + numtile
    m_s[...] = m_new

    @pl.when(i == n_tiles - 1)
    def _fin():
        out_ref[...] = num_s[...] / (d_s[...] + 1e-16) + outb_ref[...]


def kernel(x, edge_index, batch, prior, emission, gh_W, gh_b, go_W, go_b,
           out_W, out_b, contrastive):
    del edge_index
    N = x.shape[0]
    G, C = prior.shape
    M = emission.shape[2]
    P = contrastive.shape[1]
    H = gh_W.shape[1]
    F = out_W.shape[1]
    NG = 64

    TN = 2000
    assert N % TN == 0
    NT = N // TN

    x3 = x.astype(jnp.int32).reshape(NT, 1, TN)
    b3 = batch.astype(jnp.int32).reshape(NT, 1, TN)

    f32 = jnp.float32
    body = functools.partial(_cgmn_body, n_tiles=NT, n_graphs=NG)
    out = pl.pallas_call(
        body,
        grid=(NT,),
        in_specs=[
            pl.BlockSpec((1, 1, TN), lambda i: (i, 0, 0)),
            pl.BlockSpec((1, 1, TN), lambda i: (i, 0, 0)),
            pl.BlockSpec((G, C), lambda i: (0, 0)),
            pl.BlockSpec((G, C, M), lambda i: (0, 0, 0)),
            pl.BlockSpec((P, G), lambda i: (0, 0)),
            pl.BlockSpec((H, P), lambda i: (0, 0)),
            pl.BlockSpec((H, 1), lambda i: (0, 0)),
            pl.BlockSpec((H, 1), lambda i: (0, 0)),
            pl.BlockSpec((1, 1), lambda i: (0, 0)),
            pl.BlockSpec((F, P), lambda i: (0, 0)),
            pl.BlockSpec((1, F), lambda i: (0, 0)),
        ],
        out_specs=pl.BlockSpec((NG, F), lambda i: (0, 0)),
        out_shape=jax.ShapeDtypeStruct((NG, F), f32),
        scratch_shapes=[
            pltpu.VMEM((G, M), f32),
            pltpu.VMEM((NG, 1), f32),
            pltpu.VMEM((NG, 1), f32),
            pltpu.VMEM((NG, F), f32),
        ],
    )(x3, b3,
      prior.astype(f32),
      emission.astype(f32),
      contrastive.T.astype(f32),
      gh_W.T.astype(f32),
      gh_b.reshape(H, 1).astype(f32),
      go_W.reshape(H, 1).astype(f32),
      go_b.reshape(1, 1).astype(f32),
      out_W.T.astype(f32),
      out_b.reshape(1, F).astype(f32))
    return out


# speedup vs baseline: 1.9152x; 1.0902x over previous
import functools

import jax
import jax.numpy as jnp
from jax.experimental import pallas as pl
from jax.experimental.pallas import tpu as pltpu

_NEG = -1e30


def _cgmn_body(x_ref, b_ref, prior_ref, em_ref, cmT_ref, ghWT_ref,
               ghb_ref, goW_ref, gob_ref, outWT_ref, outb_ref, out_ref,
               T_s, m_s, d_s, num_s, *, n_tiles, n_graphs):
    i = pl.program_id(0)
    G, M = T_s.shape
    F = num_s.shape[1]
    TN = x_ref.shape[2]

    @pl.when(i == 0)
    def _init():
        pr = prior_ref[...]
        pe = jnp.exp(pr - jnp.max(pr, axis=1, keepdims=True))
        p = pe / jnp.sum(pe, axis=1, keepdims=True)
        em = em_ref[...]
        ee = jnp.exp(em - jnp.max(em, axis=2, keepdims=True))
        ep = ee / jnp.sum(ee, axis=2, keepdims=True)
        T_s[...] = jnp.log(jnp.sum(p[:, :, None] * ep, axis=1))
        m_s[...] = jnp.full((n_graphs, 1), _NEG, jnp.float32)
        d_s[...] = jnp.zeros((n_graphs, 1), jnp.float32)
        num_s[...] = jnp.zeros((n_graphs, F), jnp.float32)

    xi = x_ref[0]
    bi = b_ref[0]

    miota = jax.lax.broadcasted_iota(jnp.int32, (M, TN), 0)
    ohM = (miota == xi).astype(jnp.float32)
    llT = jnp.dot(T_s[...], ohM, preferred_element_type=jnp.float32)

    ciT = jnp.tanh(jnp.dot(cmT_ref[...], llT,
                           preferred_element_type=jnp.float32))
    ciT_b = ciT.astype(jnp.bfloat16)
    h = jnp.tanh(jnp.dot(ghWT_ref[...], ciT_b,
                         preferred_element_type=jnp.float32) + ghb_ref[...])
    gate = jnp.sum(h * goW_ref[...], axis=0, keepdims=True) + gob_ref[...]

    giota = jax.lax.broadcasted_iota(jnp.int32, (n_graphs, TN), 0)
    ohG = giota == bi
    ohGf = ohG.astype(jnp.float32)
    tmax = jnp.max(jnp.where(ohG, gate, _NEG), axis=1, keepdims=True)
    m_old = m_s[...]
    m_new = jnp.maximum(m_old, tmax)
    scale = jnp.exp(m_old - m_new)
    mb = jnp.sum(ohGf * m_new, axis=0, keepdims=True)
    e = jnp.exp(gate - mb)
    ohGe = ohGf * e
    d_s[...] = d_s[...] * scale + jnp.sum(ohGe, axis=1, keepdims=True)
    vT = jnp.dot(outWT_ref[...], ciT_b,
                 preferred_element_type=jnp.float32)
    numtile = jax.lax.dot_general(ohGe, vT, (((1,), (1,)), ((), ())),
                                  preferred_element_type=jnp.float32)
    num_s[...] = num_s[...] * scale + numtile
    m_s[...] = m_new

    @pl.when(i == n_tiles - 1)
    def _fin():
        out_ref[...] = num_s[...] / (d_s[...] + 1e-16) + outb_ref[...]


def kernel(x, edge_index, batch, prior, emission, gh_W, gh_b, go_W, go_b,
           out_W, out_b, contrastive):
    del edge_index
    N = x.shape[0]
    G, C = prior.shape
    M = emission.shape[2]
    P = contrastive.shape[1]
    H = gh_W.shape[1]
    F = out_W.shape[1]
    NG = 64

    TN = 2000
    assert N % TN == 0
    NT = N // TN

    x3 = x.astype(jnp.int32).reshape(NT, 1, TN)
    b3 = batch.astype(jnp.int32).reshape(NT, 1, TN)

    f32 = jnp.float32
    body = functools.partial(_cgmn_body, n_tiles=NT, n_graphs=NG)
    out = pl.pallas_call(
        body,
        grid=(NT,),
        in_specs=[
            pl.BlockSpec((1, 1, TN), lambda i: (i, 0, 0)),
            pl.BlockSpec((1, 1, TN), lambda i: (i, 0, 0)),
            pl.BlockSpec((G, C), lambda i: (0, 0)),
            pl.BlockSpec((G, C, M), lambda i: (0, 0, 0)),
            pl.BlockSpec((P, G), lambda i: (0, 0)),
            pl.BlockSpec((H, P), lambda i: (0, 0)),
            pl.BlockSpec((H, 1), lambda i: (0, 0)),
            pl.BlockSpec((H, 1), lambda i: (0, 0)),
            pl.BlockSpec((1, 1), lambda i: (0, 0)),
            pl.BlockSpec((F, P), lambda i: (0, 0)),
            pl.BlockSpec((1, F), lambda i: (0, 0)),
        ],
        out_specs=pl.BlockSpec((NG, F), lambda i: (0, 0)),
        out_shape=jax.ShapeDtypeStruct((NG, F), f32),
        scratch_shapes=[
            pltpu.VMEM((G, M), f32),
            pltpu.VMEM((NG, 1), f32),
            pltpu.VMEM((NG, 1), f32),
            pltpu.VMEM((NG, F), f32),
        ],
    )(x3, b3,
      prior.astype(f32),
      emission.astype(f32),
      contrastive.T.astype(f32),
      gh_W.T.astype(jnp.bfloat16),
      gh_b.reshape(H, 1).astype(f32),
      go_W.reshape(H, 1).astype(f32),
      go_b.reshape(1, 1).astype(f32),
      out_W.T.astype(jnp.bfloat16),
      out_b.reshape(1, F).astype(f32))
    return out
